# baseline (speedup 1.0000x reference)
import functools

import jax
import jax.numpy as jnp
from jax import lax
from jax.experimental import pallas as pl
from jax.experimental.pallas import tpu as pltpu
from jax.experimental.pallas import tpu_sc as plsc

B = 16384
L = 50
D = 64
H = 256
VOCAB = 1000000

NC = 2
NS = 16
NW = NC * NS
SAMP_PER_W = B // NW
CBLK = 4
GROWS = CBLK * L

ABLK = 65536
AGRID = -(-VOCAB // ABLK)
UPAD = AGRID * ABLK

NBLK = 16
CROWS = (B // NBLK) // 128


def _proj_body(tab_ref, w1_ref, w2_ref, u_ref):
    dw = w2_ref[:, 1] - w2_ref[:, 0]
    wrow = jnp.sum(w1_ref[...] * dw[None, :], axis=1)[None, :]
    u_ref[...] = jnp.dot(wrow, tab_ref[...],
                         preferred_element_type=jnp.float32)


_proj = pl.pallas_call(
    _proj_body,
    grid=(AGRID,),
    in_specs=[
        pl.BlockSpec((D, ABLK), lambda i: (0, i)),
        pl.BlockSpec((D, H), lambda i: (0, 0)),
        pl.BlockSpec((H, 2), lambda i: (0, 0)),
    ],
    out_specs=pl.BlockSpec((1, ABLK), lambda i: (0, i)),
    out_shape=jax.ShapeDtypeStruct((1, UPAD), jnp.float32),
)


def _sc_pool_body(xt_hbm, u_hbm, out_hbm, idx_v, dst_v, sums_v, sem):
    wid = lax.axis_index("s") * NC + lax.axis_index("c")
    for c in range(CBLK):
        pltpu.sync_copy(xt_hbm.at[:, pl.ds(wid * SAMP_PER_W + c * 128, 128)],
                        idx_v.at[pl.ds(c * L, L), :])

    def fire(j, carry):
        pltpu.async_copy(u_hbm.at[idx_v.at[j]], dst_v.at[j], sem)
        return carry

    lax.fori_loop(0, GROWS, fire, 0)

    def drain(j, carry):
        pltpu.make_async_copy(u_hbm.at[idx_v.at[j]], dst_v.at[j], sem).wait()
        return carry

    lax.fori_loop(0, GROWS, drain, 0)

    def pool(i, carry):
        c = i // 8
        lane = (i % 8) * 16
        r0 = c * L
        acc = dst_v[r0, pl.ds(lane, 16)]
        for t in range(1, L):
            acc = acc + dst_v[r0 + t, pl.ds(lane, 16)]
        sums_v[c, pl.ds(lane, 16)] = acc
        return carry

    lax.fori_loop(0, CBLK * 8, pool, 0)
    pltpu.sync_copy(sums_v, out_hbm.at[pl.ds(wid * CBLK, CBLK), :])


@functools.cache
def _sc_pool():
    return functools.partial(
        pl.kernel,
        out_type=jax.ShapeDtypeStruct((B // 128, 128), jnp.float32),
        mesh=plsc.VectorSubcoreMesh(core_axis_name="c", subcore_axis_name="s",
                                    num_cores=NC, num_subcores=NS),
        scratch_types=[
            pltpu.VMEM((GROWS, 128), jnp.int32),
            pltpu.VMEM((GROWS, 128), jnp.float32),
            pltpu.VMEM((CBLK, 128), jnp.float32),
            pltpu.SemaphoreType.DMA,
        ],
        compiler_params=pltpu.CompilerParams(use_tc_tiling_on_sc=False),
    )(_sc_pool_body)


def _loss_body(y_ref, s_ref, w2_ref, b1_ref, b2_ref, cost_ref, corr_ref):
    dw = w2_ref[:, 1] - w2_ref[:, 0]
    beta = (jnp.sum(b1_ref[0, :] * dw)
            + (b2_ref[0, 1] - b2_ref[0, 0]))
    s = s_ref[...] / jnp.float32(L) + beta
    y = y_ref[...]
    sp = jnp.where(y == 0, s, -s)
    contrib = jnp.maximum(sp, 0.0) + jnp.log1p(jnp.exp(-jnp.abs(sp)))
    cost_ref[0, 0] = jnp.sum(contrib) / jnp.float32(B)
    corr_ref[0, 0] = jnp.sum(((s > 0) == (y == 1)).astype(jnp.int32))


_loss = pl.pallas_call(
    _loss_body,
    grid=(1,),
    in_specs=[
        pl.BlockSpec((B // 128, 128), lambda i: (0, 0)),
        pl.BlockSpec((B // 128, 128), lambda i: (0, 0)),
        pl.BlockSpec((H, 2), lambda i: (0, 0)),
        pl.BlockSpec((1, H), lambda i: (0, 0)),
        pl.BlockSpec((1, 2), lambda i: (0, 0)),
    ],
    out_specs=[
        pl.BlockSpec((1, 1), lambda i: (0, 0), memory_space=pltpu.SMEM),
        pl.BlockSpec((1, 1), lambda i: (0, 0), memory_space=pltpu.SMEM),
    ],
    out_shape=[
        jax.ShapeDtypeStruct((1, 1), jnp.float32),
        jax.ShapeDtypeStruct((1, 1), jnp.int32),
    ],
)


def kernel(data_X, data_y, emb_table, W1, b1, W2, b2):
    u = _proj(emb_table.T, W1, W2).reshape(UPAD)
    sums = _sc_pool()(data_X.T, u)
    y2 = data_y.reshape(B // 128, 128)
    cost2, corr2 = _loss(y2, sums, W2, b1.reshape(1, H), b2.reshape(1, 2))
    return cost2[0, 0], corr2[0, 0]


# speedup vs baseline: 1.0215x; 1.0215x over previous
import functools

import jax
import jax.numpy as jnp
from jax import lax
from jax.experimental import pallas as pl
from jax.experimental.pallas import tpu as pltpu
from jax.experimental.pallas import tpu_sc as plsc

B = 16384
L = 50
D = 64
H = 256
VOCAB = 1000000

NC = 2
NS = 16
NW = NC * NS
SAMP_PER_W = B // NW
CBLK = 4
GROWS = CBLK * L

ABLK = 32768
AGRID = -(-VOCAB // ABLK)
UPAD = AGRID * ABLK

NBLK = 16
CROWS = (B // NBLK) // 128


def _proj_body(tab_ref, w1_ref, w2_ref, u_ref):
    dw = w2_ref[:, 1] - w2_ref[:, 0]
    wrow = jnp.sum(w1_ref[...] * dw[None, :], axis=1)[None, :]
    u_ref[...] = jnp.dot(wrow, tab_ref[...],
                         preferred_element_type=jnp.float32)


_proj = pl.pallas_call(
    _proj_body,
    grid=(AGRID,),
    in_specs=[
        pl.BlockSpec((D, ABLK), lambda i: (0, i)),
        pl.BlockSpec((D, H), lambda i: (0, 0)),
        pl.BlockSpec((H, 2), lambda i: (0, 0)),
    ],
    out_specs=pl.BlockSpec((1, ABLK), lambda i: (0, i)),
    out_shape=jax.ShapeDtypeStruct((1, UPAD), jnp.float32),
)


def _sc_pool_body(xt_hbm, u_hbm, out_hbm, idx_v, dst_v, sums_v, sem):
    wid = lax.axis_index("s") * NC + lax.axis_index("c")
    for c in range(CBLK):
        pltpu.sync_copy(xt_hbm.at[:, pl.ds(wid * SAMP_PER_W + c * 128, 128)],
                        idx_v.at[pl.ds(c * L, L), :])

    def fire(j, carry):
        pltpu.async_copy(u_hbm.at[idx_v.at[j]], dst_v.at[j], sem)
        return carry

    lax.fori_loop(0, GROWS, fire, 0)

    def drain(j, carry):
        pltpu.make_async_copy(u_hbm.at[idx_v.at[j]], dst_v.at[j], sem).wait()
        return carry

    lax.fori_loop(0, GROWS, drain, 0)

    def pool(i, carry):
        c = i // 8
        lane = (i % 8) * 16
        r0 = c * L
        acc = dst_v[r0, pl.ds(lane, 16)]
        for t in range(1, L):
            acc = acc + dst_v[r0 + t, pl.ds(lane, 16)]
        sums_v[c, pl.ds(lane, 16)] = acc
        return carry

    lax.fori_loop(0, CBLK * 8, pool, 0)
    pltpu.sync_copy(sums_v, out_hbm.at[pl.ds(wid * CBLK, CBLK), :])


@functools.cache
def _sc_pool():
    return functools.partial(
        pl.kernel,
        out_type=jax.ShapeDtypeStruct((B // 128, 128), jnp.float32),
        mesh=plsc.VectorSubcoreMesh(core_axis_name="c", subcore_axis_name="s",
                                    num_cores=NC, num_subcores=NS),
        scratch_types=[
            pltpu.VMEM((GROWS, 128), jnp.int32),
            pltpu.VMEM((GROWS, 128), jnp.float32),
            pltpu.VMEM((CBLK, 128), jnp.float32),
            pltpu.SemaphoreType.DMA,
        ],
        compiler_params=pltpu.CompilerParams(use_tc_tiling_on_sc=False),
    )(_sc_pool_body)


def _loss_body(y_ref, s_ref, w2_ref, b1_ref, b2_ref, cost_ref, corr_ref):
    dw = w2_ref[:, 1] - w2_ref[:, 0]
    beta = (jnp.sum(b1_ref[0, :] * dw)
            + (b2_ref[0, 1] - b2_ref[0, 0]))
    s = s_ref[...] / jnp.float32(L) + beta
    y = y_ref[...]
    sp = jnp.where(y == 0, s, -s)
    contrib = jnp.maximum(sp, 0.0) + jnp.log1p(jnp.exp(-jnp.abs(sp)))
    cost_ref[0, 0] = jnp.sum(contrib) / jnp.float32(B)
    corr_ref[0, 0] = jnp.sum(((s > 0) == (y == 1)).astype(jnp.int32))


_loss = pl.pallas_call(
    _loss_body,
    grid=(1,),
    in_specs=[
        pl.BlockSpec((B // 128, 128), lambda i: (0, 0)),
        pl.BlockSpec((B // 128, 128), lambda i: (0, 0)),
        pl.BlockSpec((H, 2), lambda i: (0, 0)),
        pl.BlockSpec((1, H), lambda i: (0, 0)),
        pl.BlockSpec((1, 2), lambda i: (0, 0)),
    ],
    out_specs=[
        pl.BlockSpec((1, 1), lambda i: (0, 0), memory_space=pltpu.SMEM),
        pl.BlockSpec((1, 1), lambda i: (0, 0), memory_space=pltpu.SMEM),
    ],
    out_shape=[
        jax.ShapeDtypeStruct((1, 1), jnp.float32),
        jax.ShapeDtypeStruct((1, 1), jnp.int32),
    ],
)


def kernel(data_X, data_y, emb_table, W1, b1, W2, b2):
    u = _proj(emb_table.T, W1, W2).reshape(UPAD)
    sums = _sc_pool()(data_X.T, u)
    y2 = data_y.reshape(B // 128, 128)
    cost2, corr2 = _loss(y2, sums, W2, b1.reshape(1, H), b2.reshape(1, 2))
    return cost2[0, 0], corr2[0, 0]
